# baseline (speedup 1.0000x reference)
import functools

import jax
import jax.numpy as jnp
from jax.experimental import pallas as pl

_B = 16384
_K = 1024
_E = 32
_L = 4
_BB = 256
_BETA = 0.25


def _rvq_kernel(x_ref, cb_ref, xq_ref, res_ref, loss_ref, idx_ref, dist_ref):
    @pl.when(pl.program_id(0) == 0)
    def _init():
        loss_ref[...] = jnp.zeros((1, 1), jnp.float32)

    r = x_ref[...]
    xq = jnp.zeros_like(r)
    loss_acc = jnp.float32(0.0)
    iota = jax.lax.broadcasted_iota(jnp.int32, (_BB, _K), 1)
    for lvl in range(_L):
        emb = cb_ref[lvl]
        e2 = jnp.sum(emb * emb, axis=1)
        r2 = jnp.sum(r * r, axis=1, keepdims=True)
        cross = jax.lax.dot_general(
            r, emb, (((1,), (1,)), ((), ())),
            preferred_element_type=jnp.float32)
        d = r2 + e2[None, :] - 2.0 * cross
        dist_ref[:, lvl * _K:(lvl + 1) * _K] = d
        m = jnp.min(d, axis=1, keepdims=True)
        idx = jnp.min(jnp.where(d == m, iota, _K), axis=1)
        idx_ref[lvl:lvl + 1, :] = idx[None, :]
        onehot = (iota == idx[:, None]).astype(jnp.float32)
        xq_raw = jax.lax.dot_general(
            onehot, emb, (((1,), (0,)), ((), ())),
            preferred_element_type=jnp.float32)
        diff = r - xq_raw
        loss_acc = loss_acc + jnp.sum(diff * diff)
        r = diff
        xq = xq + xq_raw
    xq_ref[...] = xq
    res_ref[...] = r
    loss_ref[...] += jnp.full((1, 1), loss_acc * ((1.0 + _BETA) / (_L * _B * _E)),
                              jnp.float32)


@functools.partial(jax.jit, static_argnames=())
def kernel(x, codebooks):
    grid = (_B // _BB,)
    xq, res, loss, idx_t, dist_flat = pl.pallas_call(
        _rvq_kernel,
        grid=grid,
        in_specs=[
            pl.BlockSpec((_BB, _E), lambda i: (i, 0)),
            pl.BlockSpec((_L, _K, _E), lambda i: (0, 0, 0)),
        ],
        out_specs=[
            pl.BlockSpec((_BB, _E), lambda i: (i, 0)),
            pl.BlockSpec((_BB, _E), lambda i: (i, 0)),
            pl.BlockSpec((1, 1), lambda i: (0, 0)),
            pl.BlockSpec((_L, _BB), lambda i: (0, i)),
            pl.BlockSpec((_BB, _L * _K), lambda i: (i, 0)),
        ],
        out_shape=[
            jax.ShapeDtypeStruct((_B, _E), jnp.float32),
            jax.ShapeDtypeStruct((_B, _E), jnp.float32),
            jax.ShapeDtypeStruct((1, 1), jnp.float32),
            jax.ShapeDtypeStruct((_L, _B), jnp.int32),
            jax.ShapeDtypeStruct((_B, _L * _K), jnp.float32),
        ],
    )(x, codebooks)
    mean_losses = loss.reshape(())
    all_indices = idx_t.T
    all_distances = dist_flat.reshape(_B, _L, _K)
    return (xq, res, mean_losses, all_indices, all_distances)


# speedup vs baseline: 1.1258x; 1.1258x over previous
import functools

import jax
import jax.numpy as jnp
from jax.experimental import pallas as pl

_B = 16384
_K = 1024
_E = 32
_L = 4
_BB = 1024
_BETA = 0.25


def _rvq_kernel(x_ref, cb_ref, xq_ref, res_ref, loss_ref, idx_ref, dist_ref):
    @pl.when(pl.program_id(0) == 0)
    def _init():
        loss_ref[...] = jnp.zeros((1, 1), jnp.float32)

    r = x_ref[...]
    xq = jnp.zeros_like(r)
    loss_acc = jnp.float32(0.0)
    iota = jax.lax.broadcasted_iota(jnp.int32, (_BB, _K), 1)
    for lvl in range(_L):
        emb = cb_ref[lvl]
        e2 = jnp.sum(emb * emb, axis=1)
        r2 = jnp.sum(r * r, axis=1, keepdims=True)
        cross = jax.lax.dot_general(
            r, emb, (((1,), (1,)), ((), ())),
            preferred_element_type=jnp.float32)
        d = r2 + e2[None, :] - 2.0 * cross
        dist_ref[:, lvl * _K:(lvl + 1) * _K] = d
        m = jnp.min(d, axis=1, keepdims=True)
        idx = jnp.min(jnp.where(d == m, iota, _K), axis=1)
        idx_ref[lvl:lvl + 1, :] = idx[None, :]
        onehot = (iota == idx[:, None]).astype(jnp.float32)
        xq_raw = jax.lax.dot_general(
            onehot, emb, (((1,), (0,)), ((), ())),
            preferred_element_type=jnp.float32)
        diff = r - xq_raw
        loss_acc = loss_acc + jnp.sum(diff * diff)
        r = diff
        xq = xq + xq_raw
    xq_ref[...] = xq
    res_ref[...] = r
    loss_ref[...] += jnp.full((1, 1), loss_acc * ((1.0 + _BETA) / (_L * _B * _E)),
                              jnp.float32)


@functools.partial(jax.jit, static_argnames=())
def kernel(x, codebooks):
    grid = (_B // _BB,)
    xq, res, loss, idx_t, dist_flat = pl.pallas_call(
        _rvq_kernel,
        grid=grid,
        in_specs=[
            pl.BlockSpec((_BB, _E), lambda i: (i, 0)),
            pl.BlockSpec((_L, _K, _E), lambda i: (0, 0, 0)),
        ],
        out_specs=[
            pl.BlockSpec((_BB, _E), lambda i: (i, 0)),
            pl.BlockSpec((_BB, _E), lambda i: (i, 0)),
            pl.BlockSpec((1, 1), lambda i: (0, 0)),
            pl.BlockSpec((_L, _BB), lambda i: (0, i)),
            pl.BlockSpec((_BB, _L * _K), lambda i: (i, 0)),
        ],
        out_shape=[
            jax.ShapeDtypeStruct((_B, _E), jnp.float32),
            jax.ShapeDtypeStruct((_B, _E), jnp.float32),
            jax.ShapeDtypeStruct((1, 1), jnp.float32),
            jax.ShapeDtypeStruct((_L, _B), jnp.int32),
            jax.ShapeDtypeStruct((_B, _L * _K), jnp.float32),
        ],
    )(x, codebooks)
    mean_losses = loss.reshape(())
    all_indices = idx_t.T
    all_distances = dist_flat.reshape(_B, _L, _K)
    return (xq, res, mean_losses, all_indices, all_distances)
